# Initial kernel scaffold; baseline (speedup 1.0000x reference)
import jax
import jax.numpy as jnp
from jax.experimental import pallas as pl


def kernel(queries, keys, k):
    raise NotImplementedError("write your pallas kernel here")


# speedup vs baseline: 5.6997x; 5.6997x over previous
import functools

import jax
import jax.numpy as jnp
from jax import lax
from jax.experimental import pallas as pl
from jax.experimental.pallas import tpu as pltpu
from jax.experimental.pallas import tpu_sc as plsc

N_KEYS = 1_000_000
D = 64
NQ = 64
K = 32

CHUNK = 8192
GSIZE = 256
GPC = CHUNK // GSIZE
NCHUNKS = (N_KEYS + CHUNK - 1) // CHUNK
NG = NCHUNKS * GPC
NG_PAD = 4096

NEG_INF = float("-inf")
BIG_I32 = 2 ** 30


def _score_kernel(q_ref, k_ref, s_ref, gmax_ref):
    c = pl.program_id(0)
    q = q_ref[...]
    qn = q / (jnp.sqrt(jnp.sum(q * q, axis=1, keepdims=True)) + 1e-10)
    kc = k_ref[...]
    sk = jnp.sum(kc * kc, axis=1)
    inv = 1.0 / (jnp.sqrt(sk) + 1e-10)
    s = lax.dot_general(qn, kc, (((1,), (1,)), ((), ())),
                        preferred_element_type=jnp.float32)
    s = s * inv[None, :]
    col = c * CHUNK + lax.broadcasted_iota(jnp.int32, (1, CHUNK), 1)
    s = jnp.where(col < N_KEYS, s, NEG_INF)
    s3 = s.reshape(NQ, GPC, GSIZE)
    s_ref[...] = s3
    gmax_ref[...] = jnp.max(s3, axis=-1).reshape(NQ, 1, 1, GPC)


def _phase_a(queries, keys):
    return pl.pallas_call(
        _score_kernel,
        grid=(NCHUNKS,),
        in_specs=[
            pl.BlockSpec((NQ, D), lambda c: (0, 0)),
            pl.BlockSpec((CHUNK, D), lambda c: (c, 0)),
        ],
        out_specs=[
            pl.BlockSpec((NQ, GPC, GSIZE), lambda c: (0, c, 0)),
            pl.BlockSpec((NQ, 1, 1, GPC), lambda c: (0, c, 0, 0)),
        ],
        out_shape=[
            jax.ShapeDtypeStruct((NQ, NG, GSIZE), jnp.float32),
            jax.ShapeDtypeStruct((NQ, NCHUNKS, 1, GPC), jnp.float32),
        ],
        compiler_params=pltpu.CompilerParams(
            dimension_semantics=("arbitrary",),
        ),
    )(queries, keys)


def _iota16():
    return lax.iota(jnp.int32, 16)


def _fsplat(x):
    return jnp.full((16,), x, dtype=jnp.float32)


def _isplat(x):
    return jnp.full((16,), x, dtype=jnp.int32)


def _lane0():
    return _iota16() == 0


def _extract_loop(nblk, load_col, mask_elem, read_m1, write_m1):
    iota = _iota16()

    def one_iter(it, carry):
        v0, v1, g0, g1 = carry

        def mx(b, acc):
            return jnp.maximum(acc, read_m1(b))
        acc = lax.fori_loop(0, nblk, mx, _fsplat(NEG_INF))
        m = jnp.max(acc)

        def fnd(b, best):
            v = read_m1(b)
            ids = iota + b * 16
            return jnp.minimum(best, jnp.where(v == m, ids, BIG_I32))
        i1 = jnp.min(lax.fori_loop(0, nblk, fnd, _isplat(BIG_I32)))
        bb = i1 // 16
        l = i1 % 16

        col = load_col(bb, l)
        jsel = jnp.min(jnp.where(col == m, iota, _isplat(BIG_I32)))
        g = bb * 256 + jsel * 16 + l

        v0 = jnp.where(iota == it, m, v0)
        v1 = jnp.where(iota == (it - 16), m, v1)
        g0 = jnp.where(iota == it, g, g0)
        g1 = jnp.where(iota == (it - 16), g, g1)

        mask_elem(bb, jsel * 16 + l)
        col2 = load_col(bb, l)
        write_m1(i1, jnp.max(col2))
        return v0, v1, g0, g1

    init = (_fsplat(NEG_INF), _fsplat(NEG_INF), _isplat(0), _isplat(0))
    return lax.fori_loop(0, K, one_iter, init)


def _select_body(scores_hbm, gmax_hbm, out_hbm,
                 gm_ref, m1_ref, cand_ref, m1b_ref, gidx_ref, outv_ref, sem):
    iota = _iota16()
    wid = lax.axis_index("s") * 2 + lax.axis_index("c")

    def per_query(t, _):
        q = wid + 32 * t

        pltpu.sync_copy(gmax_hbm.at[q], gm_ref.at[pl.ds(0, NG)])
        for j in range(NG // 16, NG_PAD // 16):
            gm_ref[pl.ds(j * 16, 16)] = _fsplat(NEG_INF)

        def build_m1(b, _):
            def mx(j, acc):
                return jnp.maximum(acc, gm_ref[pl.ds(b * 256 + j * 16, 16)])
            m1_ref[pl.ds(b * 16, 16)] = lax.fori_loop(
                0, 16, mx, _fsplat(NEG_INF))
            return 0
        lax.fori_loop(0, NG_PAD // 256, build_m1, 0)

        def g_read_m1(b):
            return m1_ref[pl.ds(b * 16, 16)]

        def g_load_col(bb, l):
            return plsc.load_gather(gm_ref, [bb * 256 + iota * 16 + l])

        def g_mask_elem(bb, off):
            plsc.store_scatter(gm_ref, [_isplat(bb * 256 + off)],
                               _fsplat(NEG_INF), mask=_lane0())

        def g_write_m1(i1, val):
            plsc.store_scatter(m1_ref, [_isplat(i1)], _fsplat(val),
                               mask=_lane0())

        _, _, grp0, grp1 = _extract_loop(
            NG_PAD // 256, g_load_col, g_mask_elem, g_read_m1, g_write_m1)

        base = q * NG
        gidx_ref[pl.ds(0, 16)] = grp0 + base
        gidx_ref[pl.ds(16, 16)] = grp1 + base
        pltpu.async_copy(scores_hbm.at[gidx_ref], cand_ref, sem).wait()

        def c_build_m1(b, _):
            def mx(j, acc):
                return jnp.maximum(
                    acc, plsc.load_gather(cand_ref,
                                          [_isplat(b), j * 16 + iota]))
            m1b_ref[pl.ds(b * 16, 16)] = lax.fori_loop(
                0, 16, mx, _fsplat(NEG_INF))
            return 0
        lax.fori_loop(0, GPC, c_build_m1, 0)

        def c_read_m1(b):
            return m1b_ref[pl.ds(b * 16, 16)]

        def c_load_col(bb, l):
            return plsc.load_gather(cand_ref, [_isplat(bb), iota * 16 + l])

        def c_mask_elem(bb, off):
            plsc.store_scatter(cand_ref, [_isplat(bb), _isplat(off)],
                               _fsplat(NEG_INF), mask=_lane0())

        def c_write_m1(i1, val):
            plsc.store_scatter(m1b_ref, [_isplat(i1)], _fsplat(val),
                               mask=_lane0())

        val0, val1, _, _ = _extract_loop(
            GPC, c_load_col, c_mask_elem, c_read_m1, c_write_m1)

        outv_ref[pl.ds(0, 16)] = val0
        outv_ref[pl.ds(16, 16)] = val1
        pltpu.sync_copy(outv_ref, out_hbm.at[q])
        return 0

    lax.fori_loop(0, 2, per_query, 0)


def _phase_b(scores_flat, gmax):
    mesh = plsc.VectorSubcoreMesh(core_axis_name="c", subcore_axis_name="s",
                                  num_cores=2, num_subcores=16)
    fn = pl.kernel(
        _select_body,
        out_type=jax.ShapeDtypeStruct((NQ, K), jnp.float32),
        mesh=mesh,
        scratch_types=[
            pltpu.VMEM((NG_PAD,), jnp.float32),
            pltpu.VMEM((NG_PAD // 16,), jnp.float32),
            pltpu.VMEM((K, GSIZE), jnp.float32),
            pltpu.VMEM((GPC * 16,), jnp.float32),
            pltpu.VMEM((K,), jnp.int32),
            pltpu.VMEM((K,), jnp.float32),
            pltpu.SemaphoreType.DMA,
        ],
        compiler_params=pltpu.CompilerParams(needs_layout_passes=False,
                                             use_tc_tiling_on_sc=False),
    )
    return fn(scores_flat, gmax)


def kernel(queries, keys, k):
    scores3, gmax4 = _phase_a(queries, keys)
    scores_flat = scores3.reshape(NQ * NG, GSIZE)
    gmax = gmax4.reshape(NQ, NG)
    return _phase_b(scores_flat, gmax)
